# baseline (speedup 1.0000x reference)
import jax
import jax.numpy as jnp
from jax import lax
from jax.experimental import pallas as pl
from jax.experimental.pallas import tpu as pltpu
from jax.experimental.pallas import tpu_sc as plsc

BATCH = 16384
LATENT = 64
PADW = 128
ROWS = 1000000
NC = 2
NS = 16
NW = NC * NS
B_PER_W = BATCH // NW
CHUNK = 128
NCHUNK = B_PER_W // CHUNK
TBLK = 8192


def _concat_body(ut_ref, it_ref, out_ref):
    out_ref[...] = jnp.concatenate(
        [jnp.transpose(ut_ref[...], (1, 0)),
         jnp.transpose(it_ref[...], (1, 0))], axis=1)


def _tc_concat(ut_t, it_t):
    grid = (pl.cdiv(ROWS, TBLK),)
    return pl.pallas_call(
        _concat_body,
        grid=grid,
        in_specs=[
            pl.BlockSpec((LATENT, TBLK), lambda i: (0, i)),
            pl.BlockSpec((LATENT, TBLK), lambda i: (0, i)),
        ],
        out_specs=pl.BlockSpec((TBLK, PADW), lambda i: (i, 0)),
        out_shape=jax.ShapeDtypeStruct((ROWS, PADW), jnp.float32),
        compiler_params=pltpu.CompilerParams(
            dimension_semantics=("parallel",)),
    )(ut_t, it_t)


def _sc_gather_kernel(tbl_hbm, uidx_hbm, iidx_hbm, uout_hbm, iout_hbm,
                      uidx_v, iidx_v, rows_v, sem):
    wid = lax.axis_index("s") * NC + lax.axis_index("c")
    base = wid * B_PER_W
    rbase = wid * NCHUNK

    pltpu.sync_copy(uidx_hbm.at[pl.ds(rbase, NCHUNK)], uidx_v)
    pltpu.sync_copy(iidx_hbm.at[pl.ds(rbase, NCHUNK)], iidx_v)

    for idx_v, out_hbm in ((uidx_v, uout_hbm), (iidx_v, iout_hbm)):
        copies = []
        for j in range(NCHUNK):
            copies.append(
                pltpu.async_copy(tbl_hbm.at[idx_v.at[j]],
                                 rows_v.at[pl.ds(j * CHUNK, CHUNK)], sem))
        for c in copies:
            c.wait()
        pltpu.sync_copy(rows_v, out_hbm.at[pl.ds(base, B_PER_W)])


def _sc_gather(tbl, user_idx, item_idx):
    mesh = plsc.VectorSubcoreMesh(core_axis_name="c", subcore_axis_name="s")
    f32 = jnp.float32
    uidx2d = user_idx.reshape(NW * NCHUNK, CHUNK)
    iidx2d = item_idx.reshape(NW * NCHUNK, CHUNK)
    kern = pl.kernel(
        _sc_gather_kernel,
        out_type=[
            jax.ShapeDtypeStruct((BATCH, PADW), f32),
            jax.ShapeDtypeStruct((BATCH, PADW), f32),
        ],
        mesh=mesh,
        scratch_types=[
            pltpu.VMEM((NCHUNK, CHUNK), jnp.int32),
            pltpu.VMEM((NCHUNK, CHUNK), jnp.int32),
            pltpu.VMEM((B_PER_W, PADW), f32),
            pltpu.SemaphoreType.DMA,
        ],
    )
    return kern(tbl, uidx2d, iidx2d)


def _mlp_body(u_ref, v_ref, wh_ref, bh_ref, wg_ref, bg_ref, hid_ref, score_ref):
    bf16 = jnp.bfloat16
    f32 = jnp.float32
    sim = u_ref[:, :LATENT] * v_ref[:, LATENT:]
    hid = lax.dot_general(sim.astype(bf16), wh_ref[...].astype(bf16),
                          dimension_numbers=(((1,), (1,)), ((), ())),
                          preferred_element_type=f32)
    hid = hid + bh_ref[...]
    hid_ref[...] = hid
    hb = hid.astype(bf16).astype(f32)
    wb = wg_ref[...].astype(bf16).astype(f32)
    score_ref[...] = (jnp.sum(hb * wb, axis=1, keepdims=True)
                      + bg_ref[0, 0])


def _tc_mlp(u_rows, v_rows, W_hidden, b_hidden, W_gmf, b_gmf):
    blk = 2048
    grid = (BATCH // blk,)
    f32 = jnp.float32
    hid, score = pl.pallas_call(
        _mlp_body,
        grid=grid,
        in_specs=[
            pl.BlockSpec((blk, PADW), lambda i: (i, 0)),
            pl.BlockSpec((blk, PADW), lambda i: (i, 0)),
            pl.BlockSpec((LATENT // 2, LATENT), lambda i: (0, 0)),
            pl.BlockSpec((1, LATENT // 2), lambda i: (0, 0)),
            pl.BlockSpec((1, LATENT // 2), lambda i: (0, 0)),
            pl.BlockSpec((1, 1), lambda i: (0, 0)),
        ],
        out_specs=[
            pl.BlockSpec((blk, LATENT // 2), lambda i: (i, 0)),
            pl.BlockSpec((blk, 1), lambda i: (i, 0)),
        ],
        out_shape=[
            jax.ShapeDtypeStruct((BATCH, LATENT // 2), f32),
            jax.ShapeDtypeStruct((BATCH, 1), f32),
        ],
    )(u_rows, v_rows, W_hidden, b_hidden.reshape(1, -1), W_gmf,
      b_gmf.reshape(1, 1))
    return hid, score


def kernel(user, item, user_table, item_table, W_hidden, b_hidden, W_gmf, b_gmf):
    tbl = _tc_concat(user_table.T, item_table.T)
    u_rows, v_rows = _sc_gather(tbl, user, item)
    hid, score = _tc_mlp(u_rows, v_rows, W_hidden, b_hidden, W_gmf, b_gmf)
    return (score, hid)


# speedup vs baseline: 1.0652x; 1.0652x over previous
import jax
import jax.numpy as jnp
from jax import lax
from jax.experimental import pallas as pl
from jax.experimental.pallas import tpu as pltpu
from jax.experimental.pallas import tpu_sc as plsc

BATCH = 16384
LATENT = 64
PADW = 128
ROWS = 1000000
NC = 2
NS = 16
NW = NC * NS
B_PER_W = BATCH // NW
CHUNK = 128
NCHUNK = B_PER_W // CHUNK
TBLK = 16384


def _concat_body(ut_ref, it_ref, out_ref):
    out_ref[...] = jnp.concatenate(
        [jnp.transpose(ut_ref[...], (1, 0)),
         jnp.transpose(it_ref[...], (1, 0))], axis=1)


def _tc_concat(ut_t, it_t):
    grid = (pl.cdiv(ROWS, TBLK),)
    return pl.pallas_call(
        _concat_body,
        grid=grid,
        in_specs=[
            pl.BlockSpec((LATENT, TBLK), lambda i: (0, i)),
            pl.BlockSpec((LATENT, TBLK), lambda i: (0, i)),
        ],
        out_specs=pl.BlockSpec((TBLK, PADW), lambda i: (i, 0)),
        out_shape=jax.ShapeDtypeStruct((ROWS, PADW), jnp.float32),
        compiler_params=pltpu.CompilerParams(
            dimension_semantics=("parallel",)),
    )(ut_t, it_t)


def _sc_gather_kernel(tbl_hbm, uidx_hbm, iidx_hbm, uout_hbm, iout_hbm,
                      uidx_v, iidx_v, rows_v, sem):
    wid = lax.axis_index("s") * NC + lax.axis_index("c")
    base = wid * B_PER_W
    rbase = wid * NCHUNK

    pltpu.sync_copy(uidx_hbm.at[pl.ds(rbase, NCHUNK)], uidx_v)
    pltpu.sync_copy(iidx_hbm.at[pl.ds(rbase, NCHUNK)], iidx_v)

    for idx_v, out_hbm in ((uidx_v, uout_hbm), (iidx_v, iout_hbm)):
        copies = []
        for j in range(NCHUNK):
            copies.append(
                pltpu.async_copy(tbl_hbm.at[idx_v.at[j]],
                                 rows_v.at[pl.ds(j * CHUNK, CHUNK)], sem))
        for c in copies:
            c.wait()
        pltpu.sync_copy(rows_v, out_hbm.at[pl.ds(base, B_PER_W)])


def _sc_gather(tbl, user_idx, item_idx):
    mesh = plsc.VectorSubcoreMesh(core_axis_name="c", subcore_axis_name="s")
    f32 = jnp.float32
    uidx2d = user_idx.reshape(NW * NCHUNK, CHUNK)
    iidx2d = item_idx.reshape(NW * NCHUNK, CHUNK)
    kern = pl.kernel(
        _sc_gather_kernel,
        out_type=[
            jax.ShapeDtypeStruct((BATCH, PADW), f32),
            jax.ShapeDtypeStruct((BATCH, PADW), f32),
        ],
        mesh=mesh,
        scratch_types=[
            pltpu.VMEM((NCHUNK, CHUNK), jnp.int32),
            pltpu.VMEM((NCHUNK, CHUNK), jnp.int32),
            pltpu.VMEM((B_PER_W, PADW), f32),
            pltpu.SemaphoreType.DMA,
        ],
    )
    return kern(tbl, uidx2d, iidx2d)


def _mlp_body(u_ref, v_ref, wh_ref, bh_ref, wg_ref, bg_ref, hid_ref, score_ref):
    bf16 = jnp.bfloat16
    f32 = jnp.float32
    sim = u_ref[:, :LATENT] * v_ref[:, LATENT:]
    hid = lax.dot_general(sim.astype(bf16), wh_ref[...].astype(bf16),
                          dimension_numbers=(((1,), (1,)), ((), ())),
                          preferred_element_type=f32)
    hid = hid + bh_ref[...]
    hid_ref[...] = hid
    hb = hid.astype(bf16).astype(f32)
    wb = wg_ref[...].astype(bf16).astype(f32)
    score_ref[...] = (jnp.sum(hb * wb, axis=1, keepdims=True)
                      + bg_ref[0, 0])


def _tc_mlp(u_rows, v_rows, W_hidden, b_hidden, W_gmf, b_gmf):
    blk = 2048
    grid = (BATCH // blk,)
    f32 = jnp.float32
    hid, score = pl.pallas_call(
        _mlp_body,
        grid=grid,
        in_specs=[
            pl.BlockSpec((blk, PADW), lambda i: (i, 0)),
            pl.BlockSpec((blk, PADW), lambda i: (i, 0)),
            pl.BlockSpec((LATENT // 2, LATENT), lambda i: (0, 0)),
            pl.BlockSpec((1, LATENT // 2), lambda i: (0, 0)),
            pl.BlockSpec((1, LATENT // 2), lambda i: (0, 0)),
            pl.BlockSpec((1, 1), lambda i: (0, 0)),
        ],
        out_specs=[
            pl.BlockSpec((blk, LATENT // 2), lambda i: (i, 0)),
            pl.BlockSpec((blk, 1), lambda i: (i, 0)),
        ],
        out_shape=[
            jax.ShapeDtypeStruct((BATCH, LATENT // 2), f32),
            jax.ShapeDtypeStruct((BATCH, 1), f32),
        ],
    )(u_rows, v_rows, W_hidden, b_hidden.reshape(1, -1), W_gmf,
      b_gmf.reshape(1, 1))
    return hid, score


def kernel(user, item, user_table, item_table, W_hidden, b_hidden, W_gmf, b_gmf):
    tbl = _tc_concat(user_table.T, item_table.T)
    u_rows, v_rows = _sc_gather(tbl, user, item)
    hid, score = _tc_mlp(u_rows, v_rows, W_hidden, b_hidden, W_gmf, b_gmf)
    return (score, hid)


# speedup vs baseline: 1.0950x; 1.0280x over previous
import jax
import jax.numpy as jnp
from jax import lax
from jax.experimental import pallas as pl
from jax.experimental.pallas import tpu as pltpu
from jax.experimental.pallas import tpu_sc as plsc

BATCH = 16384
LATENT = 64
PADW = 128
ROWS = 1000000
NC = 2
NS = 16
NW = NC * NS
B_PER_W = BATCH // NW
CHUNK = 128
NCHUNK = B_PER_W // CHUNK
TBLK = 16384


def _concat_body(ut_ref, it_ref, out_ref):
    out_ref[...] = jnp.concatenate(
        [jnp.transpose(ut_ref[...], (1, 0)),
         jnp.transpose(it_ref[...], (1, 0))], axis=1)


def _tc_concat(ut_t, it_t):
    grid = (pl.cdiv(ROWS, TBLK),)
    return pl.pallas_call(
        _concat_body,
        grid=grid,
        in_specs=[
            pl.BlockSpec((LATENT, TBLK), lambda i: (0, i)),
            pl.BlockSpec((LATENT, TBLK), lambda i: (0, i)),
        ],
        out_specs=pl.BlockSpec((TBLK, PADW), lambda i: (i, 0)),
        out_shape=jax.ShapeDtypeStruct((ROWS, PADW), jnp.float32),
        compiler_params=pltpu.CompilerParams(
            dimension_semantics=("parallel",)),
    )(ut_t, it_t)


def _sc_gather_kernel(tbl_hbm, uidx_hbm, iidx_hbm, uout_hbm, iout_hbm,
                      uidx_v, iidx_v, rows_v, sem):
    wid = lax.axis_index("s") * NC + lax.axis_index("c")
    base = wid * B_PER_W
    rbase = wid * NCHUNK

    pltpu.sync_copy(uidx_hbm.at[pl.ds(rbase, NCHUNK)], uidx_v)
    pltpu.sync_copy(iidx_hbm.at[pl.ds(rbase, NCHUNK)], iidx_v)

    for idx_v, out_hbm in ((uidx_v, uout_hbm), (iidx_v, iout_hbm)):
        copies = []
        for j in range(NCHUNK):
            copies.append(
                pltpu.async_copy(tbl_hbm.at[idx_v.at[j]],
                                 rows_v.at[pl.ds(j * CHUNK, CHUNK)], sem))
        for c in copies:
            c.wait()
        pltpu.sync_copy(rows_v, out_hbm.at[pl.ds(base, B_PER_W)])


def _sc_gather(tbl, user_idx, item_idx):
    mesh = plsc.VectorSubcoreMesh(core_axis_name="c", subcore_axis_name="s")
    f32 = jnp.float32
    uidx2d = user_idx.reshape(NW * NCHUNK, CHUNK)
    iidx2d = item_idx.reshape(NW * NCHUNK, CHUNK)
    kern = pl.kernel(
        _sc_gather_kernel,
        out_type=[
            jax.ShapeDtypeStruct((BATCH, PADW), f32),
            jax.ShapeDtypeStruct((BATCH, PADW), f32),
        ],
        mesh=mesh,
        scratch_types=[
            pltpu.VMEM((NCHUNK, CHUNK), jnp.int32),
            pltpu.VMEM((NCHUNK, CHUNK), jnp.int32),
            pltpu.VMEM((B_PER_W, PADW), f32),
            pltpu.SemaphoreType.DMA,
        ],
    )
    return kern(tbl, uidx2d, iidx2d)


def _mlp_body(u_ref, v_ref, wh_ref, bh_ref, wg_ref, bg_ref, hid_ref, score_ref):
    bf16 = jnp.bfloat16
    f32 = jnp.float32
    sim = u_ref[:, :LATENT] * v_ref[:, LATENT:]
    hid = lax.dot_general(sim.astype(bf16), wh_ref[...].astype(bf16),
                          dimension_numbers=(((1,), (1,)), ((), ())),
                          preferred_element_type=f32)
    hid = hid + bh_ref[...]
    hid_ref[...] = jnp.transpose(hid, (1, 0))
    hb = hid.astype(bf16).astype(f32)
    wb = wg_ref[...].astype(bf16).astype(f32)
    score = jnp.sum(hb * wb, axis=1, keepdims=True) + bg_ref[0, 0]
    score_ref[...] = jnp.transpose(score, (1, 0))


def _tc_mlp(u_rows, v_rows, W_hidden, b_hidden, W_gmf, b_gmf):
    blk = 2048
    grid = (BATCH // blk,)
    f32 = jnp.float32
    hid, score = pl.pallas_call(
        _mlp_body,
        grid=grid,
        in_specs=[
            pl.BlockSpec((blk, PADW), lambda i: (i, 0)),
            pl.BlockSpec((blk, PADW), lambda i: (i, 0)),
            pl.BlockSpec((LATENT // 2, LATENT), lambda i: (0, 0)),
            pl.BlockSpec((1, LATENT // 2), lambda i: (0, 0)),
            pl.BlockSpec((1, LATENT // 2), lambda i: (0, 0)),
            pl.BlockSpec((1, 1), lambda i: (0, 0)),
        ],
        out_specs=[
            pl.BlockSpec((LATENT // 2, blk), lambda i: (0, i)),
            pl.BlockSpec((1, blk), lambda i: (0, i)),
        ],
        out_shape=[
            jax.ShapeDtypeStruct((LATENT // 2, BATCH), f32),
            jax.ShapeDtypeStruct((1, BATCH), f32),
        ],
    )(u_rows, v_rows, W_hidden, b_hidden.reshape(1, -1), W_gmf,
      b_gmf.reshape(1, 1))
    return hid.T, score.T


def kernel(user, item, user_table, item_table, W_hidden, b_hidden, W_gmf, b_gmf):
    tbl = _tc_concat(user_table.T, item_table.T)
    u_rows, v_rows = _sc_gather(tbl, user, item)
    hid, score = _tc_mlp(u_rows, v_rows, W_hidden, b_hidden, W_gmf, b_gmf)
    return (score, hid)


# speedup vs baseline: 1.0955x; 1.0004x over previous
import jax
import jax.numpy as jnp
from jax import lax
from jax.experimental import pallas as pl
from jax.experimental.pallas import tpu as pltpu
from jax.experimental.pallas import tpu_sc as plsc

BATCH = 16384
LATENT = 64
PADW = 128
ROWS = 1000000
NC = 2
NS = 16
NW = NC * NS
B_PER_W = BATCH // NW
CHUNK = 128
NCHUNK = B_PER_W // CHUNK
TBLK = 16384


def _concat_body(ut_ref, it_ref, out_ref):
    out_ref[:, :LATENT] = jnp.transpose(ut_ref[...], (1, 0))
    out_ref[:, LATENT:] = jnp.transpose(it_ref[...], (1, 0))


def _tc_concat(ut_t, it_t):
    grid = (pl.cdiv(ROWS, TBLK),)
    return pl.pallas_call(
        _concat_body,
        grid=grid,
        in_specs=[
            pl.BlockSpec((LATENT, TBLK), lambda i: (0, i)),
            pl.BlockSpec((LATENT, TBLK), lambda i: (0, i)),
        ],
        out_specs=pl.BlockSpec((TBLK, PADW), lambda i: (i, 0)),
        out_shape=jax.ShapeDtypeStruct((ROWS, PADW), jnp.float32),
        compiler_params=pltpu.CompilerParams(
            dimension_semantics=("parallel",)),
    )(ut_t, it_t)


def _sc_gather_kernel(tbl_hbm, uidx_hbm, iidx_hbm, uout_hbm, iout_hbm,
                      uidx_v, iidx_v, rows_v, sem):
    wid = lax.axis_index("s") * NC + lax.axis_index("c")
    base = wid * B_PER_W
    rbase = wid * NCHUNK

    pltpu.sync_copy(uidx_hbm.at[pl.ds(rbase, NCHUNK)], uidx_v)
    pltpu.sync_copy(iidx_hbm.at[pl.ds(rbase, NCHUNK)], iidx_v)

    for idx_v, out_hbm in ((uidx_v, uout_hbm), (iidx_v, iout_hbm)):
        copies = []
        for j in range(NCHUNK):
            copies.append(
                pltpu.async_copy(tbl_hbm.at[idx_v.at[j]],
                                 rows_v.at[pl.ds(j * CHUNK, CHUNK)], sem))
        for c in copies:
            c.wait()
        pltpu.sync_copy(rows_v, out_hbm.at[pl.ds(base, B_PER_W)])


def _sc_gather(tbl, user_idx, item_idx):
    mesh = plsc.VectorSubcoreMesh(core_axis_name="c", subcore_axis_name="s")
    f32 = jnp.float32
    uidx2d = user_idx.reshape(NW * NCHUNK, CHUNK)
    iidx2d = item_idx.reshape(NW * NCHUNK, CHUNK)
    kern = pl.kernel(
        _sc_gather_kernel,
        out_type=[
            jax.ShapeDtypeStruct((BATCH, PADW), f32),
            jax.ShapeDtypeStruct((BATCH, PADW), f32),
        ],
        mesh=mesh,
        scratch_types=[
            pltpu.VMEM((NCHUNK, CHUNK), jnp.int32),
            pltpu.VMEM((NCHUNK, CHUNK), jnp.int32),
            pltpu.VMEM((B_PER_W, PADW), f32),
            pltpu.SemaphoreType.DMA,
        ],
    )
    return kern(tbl, uidx2d, iidx2d)


def _mlp_body(u_ref, v_ref, wh_ref, bh_ref, wg_ref, bg_ref, hid_ref, score_ref):
    bf16 = jnp.bfloat16
    f32 = jnp.float32
    sim = u_ref[:, :LATENT] * v_ref[:, LATENT:]
    hid = lax.dot_general(sim.astype(bf16), wh_ref[...].astype(bf16),
                          dimension_numbers=(((1,), (1,)), ((), ())),
                          preferred_element_type=f32)
    hid = hid + bh_ref[...]
    hid_ref[...] = jnp.transpose(hid, (1, 0))
    hb = hid.astype(bf16).astype(f32)
    wb = wg_ref[...].astype(bf16).astype(f32)
    score = jnp.sum(hb * wb, axis=1, keepdims=True) + bg_ref[0, 0]
    score_ref[...] = jnp.transpose(score, (1, 0))


def _tc_mlp(u_rows, v_rows, W_hidden, b_hidden, W_gmf, b_gmf):
    blk = 2048
    grid = (BATCH // blk,)
    f32 = jnp.float32
    hid, score = pl.pallas_call(
        _mlp_body,
        grid=grid,
        in_specs=[
            pl.BlockSpec((blk, PADW), lambda i: (i, 0)),
            pl.BlockSpec((blk, PADW), lambda i: (i, 0)),
            pl.BlockSpec((LATENT // 2, LATENT), lambda i: (0, 0)),
            pl.BlockSpec((1, LATENT // 2), lambda i: (0, 0)),
            pl.BlockSpec((1, LATENT // 2), lambda i: (0, 0)),
            pl.BlockSpec((1, 1), lambda i: (0, 0)),
        ],
        out_specs=[
            pl.BlockSpec((LATENT // 2, blk), lambda i: (0, i)),
            pl.BlockSpec((1, blk), lambda i: (0, i)),
        ],
        out_shape=[
            jax.ShapeDtypeStruct((LATENT // 2, BATCH), f32),
            jax.ShapeDtypeStruct((1, BATCH), f32),
        ],
    )(u_rows, v_rows, W_hidden, b_hidden.reshape(1, -1), W_gmf,
      b_gmf.reshape(1, 1))
    return hid.T, score.T


def kernel(user, item, user_table, item_table, W_hidden, b_hidden, W_gmf, b_gmf):
    tbl = _tc_concat(user_table.T, item_table.T)
    u_rows, v_rows = _sc_gather(tbl, user, item)
    hid, score = _tc_mlp(u_rows, v_rows, W_hidden, b_hidden, W_gmf, b_gmf)
    return (score, hid)
